# baseline (speedup 1.0000x reference)
import functools

import jax
import jax.numpy as jnp
from jax import lax
from jax.experimental import pallas as pl
from jax.experimental.pallas import tpu as pltpu
from jax.experimental.pallas import tpu_sc as plsc

D = 128
CH = 100
NC = 2
NS = 16
NW = NC * NS


@functools.lru_cache(maxsize=None)
def _build(B, T):
    rpw = B // NW
    nix = T // CH
    mesh = plsc.VectorSubcoreMesh(core_axis_name="c", subcore_axis_name="s")

    @functools.partial(
        pl.kernel,
        mesh=mesh,
        out_type=jax.ShapeDtypeStruct((B, T, D), jnp.float32),
        scratch_types=[
            pltpu.VMEM((rpw * nix, CH), jnp.int32),
            pltpu.VMEM((2, T, D), jnp.float32),
            pltpu.SemaphoreType.DMA,
            pltpu.SemaphoreType.DMA,
            pltpu.SemaphoreType.DMA,
            pltpu.SemaphoreType.DMA,
            pltpu.SemaphoreType.DMA,
            pltpu.SemaphoreType.DMA,
        ],
    )
    def emb(x_hbm, tok_hbm, pos_hbm, out_hbm, idx_v, rows_v,
            sg0, sg1, so0, so1, sp0, sp1):
        wid = lax.axis_index("s") * NC + lax.axis_index("c")
        base = wid * rpw
        sgs = (sg0, sg1)
        sos = (so0, so1)
        sps = (sp0, sp1)

        pltpu.sync_copy(x_hbm.at[pl.ds(base * nix, rpw * nix)], idx_v)

        def start_row(jj, b):
            pltpu.async_copy(pos_hbm, rows_v.at[b], sps[b]).wait()
            for c in range(nix):
                pltpu.async_copy(tok_hbm.at[idx_v.at[jj * nix + c]],
                                 rows_v.at[b].at[pl.ds(c * CH, CH)], sgs[b],
                                 add=True)

        def wait_gathers(b):
            for c in range(nix):
                pltpu.make_async_copy(tok_hbm.at[idx_v.at[c]],
                                      rows_v.at[b].at[pl.ds(c * CH, CH)],
                                      sgs[b]).wait()

        def start_out(jj, b):
            pltpu.async_copy(rows_v.at[b], out_hbm.at[base + jj], sos[b])

        def wait_out(b):
            pltpu.make_async_copy(rows_v.at[b], out_hbm.at[base], sos[b]).wait()

        start_row(0, 0)

        def body(j2, carry):
            for b in range(2):
                jj = 2 * j2 + b
                nb = 1 - b
                have_next = jj + 1 < rpw
                if b == 0:
                    can_wait = jnp.logical_and(have_next, j2 >= 1)
                else:
                    can_wait = have_next

                @pl.when(can_wait)
                def _():
                    wait_out(nb)

                @pl.when(have_next)
                def _():
                    start_row(jj + 1, nb)

                wait_gathers(b)
                start_out(jj, b)
            return carry

        lax.fori_loop(0, rpw // 2, body, 0)
        wait_out(0)
        wait_out(1)

    return emb


def kernel(x, token_table, pos_table):
    B, T = x.shape
    x2 = x.reshape((B * T) // CH, CH).astype(jnp.int32)
    return _build(B, T)(x2, token_table, pos_table)


# speedup vs baseline: 1.4855x; 1.4855x over previous
import functools

import jax
import jax.numpy as jnp
from jax import lax
from jax.experimental import pallas as pl
from jax.experimental.pallas import tpu as pltpu
from jax.experimental.pallas import tpu_sc as plsc

D = 128
SU = (96, 104)
NB = 4
PF = 2
NC = 2
NS = 16
NW = NC * NS


@functools.lru_cache(maxsize=None)
def _build(B, T):
    rpw = B // NW
    nu = rpw * 2
    ntok = rpw * T
    mesh = plsc.VectorSubcoreMesh(core_axis_name="c", subcore_axis_name="s")

    @functools.partial(
        pl.kernel,
        mesh=mesh,
        out_type=jax.ShapeDtypeStruct((B, T, D), jnp.float32),
        scratch_types=[
            pltpu.VMEM((ntok,), jnp.int32),
            pltpu.VMEM((NB, SU[1], D), jnp.float32),
            pltpu.VMEM((T, D), jnp.float32),
            pltpu.SemaphoreType.DMA,
            pltpu.SemaphoreType.DMA,
            pltpu.SemaphoreType.DMA,
            pltpu.SemaphoreType.DMA,
            pltpu.SemaphoreType.DMA,
            pltpu.SemaphoreType.DMA,
            pltpu.SemaphoreType.DMA,
            pltpu.SemaphoreType.DMA,
        ],
    )
    def emb(x_hbm, tok_hbm, pos_hbm, out_hbm, idx_v, rows_v, pos_v,
            sg0, sg1, sg2, sg3, so0, so1, so2, so3):
        wid = lax.axis_index("s") * NC + lax.axis_index("c")
        tok0 = wid * ntok
        row0 = wid * rpw
        sgs = (sg0, sg1, sg2, sg3)
        sos = (so0, so1, so2, so3)

        pltpu.sync_copy(pos_hbm, pos_v)
        pltpu.sync_copy(x_hbm.at[pl.ds(tok0, ntok)], idx_v)

        def start_gather(k, b):
            su = SU[b % 2]
            off = (2 * k + b // 2) * T + (b % 2) * SU[0]
            pltpu.async_copy(tok_hbm.at[idx_v.at[pl.ds(off, su)]],
                             rows_v.at[b].at[pl.ds(0, su)], sgs[b])

        def wait_gather(b):
            su = SU[b % 2]
            pltpu.make_async_copy(tok_hbm.at[idx_v.at[pl.ds(0, su)]],
                                  rows_v.at[b].at[pl.ds(0, su)],
                                  sgs[b]).wait()

        def start_out(k, b):
            su = SU[b % 2]
            brow = row0 + 2 * k + b // 2
            pltpu.async_copy(rows_v.at[b].at[pl.ds(0, su)],
                             out_hbm.at[brow].at[pl.ds((b % 2) * SU[0], su)],
                             sos[b])

        def wait_out(b):
            su = SU[b % 2]
            pltpu.make_async_copy(rows_v.at[b].at[pl.ds(0, su)],
                                  out_hbm.at[0].at[pl.ds(0, su)],
                                  sos[b]).wait()

        def add_pos(b):
            su = SU[b % 2]
            t0 = (b % 2) * SU[0]

            def row(i, carry):
                for d in range(D // 16):
                    s0 = pl.ds(d * 16, 16)
                    rows_v[b, i, s0] = rows_v[b, i, s0] + pos_v[t0 + i, s0]
                return carry

            lax.fori_loop(0, su, row, 0, unroll=4)

        for b in range(PF):
            start_gather(0, b)

        def body(k, carry):
            for b in range(NB):
                u = NB * k + b
                pb = (b + PF) % NB
                have_next = u + PF < nu
                if b + PF < NB:
                    can_wait = jnp.logical_and(have_next, k >= 1)
                else:
                    can_wait = have_next

                @pl.when(can_wait)
                def _():
                    wait_out(pb)

                @pl.when(have_next)
                def _():
                    start_gather(k + (b + PF) // NB, pb)

                wait_gather(b)
                add_pos(b)
                start_out(k, b)
            return carry

        lax.fori_loop(0, nu // NB, body, 0)
        for b in range(NB):
            wait_out(b)

    return emb


def kernel(x, token_table, pos_table):
    B, T = x.shape
    xf = x.reshape(B * T).astype(jnp.int32)
    return _build(B, T)(xf, token_table, pos_table)


# speedup vs baseline: 2.8125x; 1.8933x over previous
import functools

import jax
import jax.numpy as jnp
from jax import lax
from jax.experimental import pallas as pl
from jax.experimental.pallas import tpu as pltpu
from jax.experimental.pallas import tpu_sc as plsc

D = 128
SU = (96, 104)
NB = 4
PF = 2
NC = 2
NS = 16
NW = NC * NS


@functools.lru_cache(maxsize=None)
def _build(B, T):
    rpw = B // NW
    nu = rpw * 2
    ntok = rpw * T
    mesh = plsc.VectorSubcoreMesh(core_axis_name="c", subcore_axis_name="s")

    @functools.partial(
        pl.kernel,
        mesh=mesh,
        out_type=jax.ShapeDtypeStruct((B, T, D), jnp.float32),
        scratch_types=[
            pltpu.VMEM((ntok,), jnp.int32),
            pltpu.VMEM((NB, SU[1], D), jnp.float32),
            pltpu.VMEM((T, D), jnp.float32),
            pltpu.SemaphoreType.DMA,
            pltpu.SemaphoreType.DMA,
            pltpu.SemaphoreType.DMA,
            pltpu.SemaphoreType.DMA,
            pltpu.SemaphoreType.DMA,
            pltpu.SemaphoreType.DMA,
            pltpu.SemaphoreType.DMA,
            pltpu.SemaphoreType.DMA,
        ],
    )
    def emb(x_hbm, tok_hbm, pos_hbm, out_hbm, idx_v, rows_v, pos_v,
            sg0, sg1, sg2, sg3, so0, so1, so2, so3):
        wid = lax.axis_index("s") * NC + lax.axis_index("c")
        tok0 = wid * ntok
        row0 = wid * rpw
        sgs = (sg0, sg1, sg2, sg3)
        sos = (so0, so1, so2, so3)

        pltpu.sync_copy(pos_hbm, pos_v)
        pltpu.sync_copy(x_hbm.at[pl.ds(tok0, ntok)], idx_v)

        def start_gather(k, b):
            su = SU[b % 2]
            off = (2 * k + b // 2) * T + (b % 2) * SU[0]
            pltpu.async_copy(tok_hbm.at[idx_v.at[pl.ds(off, su)]],
                             rows_v.at[b].at[pl.ds(0, su)], sgs[b])

        def wait_gather(b):
            su = SU[b % 2]
            pltpu.make_async_copy(tok_hbm.at[idx_v.at[pl.ds(0, su)]],
                                  rows_v.at[b].at[pl.ds(0, su)],
                                  sgs[b]).wait()

        def start_out(k, b):
            su = SU[b % 2]
            brow = row0 + 2 * k + b // 2
            pltpu.async_copy(rows_v.at[b].at[pl.ds(0, su)],
                             out_hbm.at[brow].at[pl.ds((b % 2) * SU[0], su)],
                             sos[b])

        def wait_out(b):
            su = SU[b % 2]
            pltpu.make_async_copy(rows_v.at[b].at[pl.ds(0, su)],
                                  out_hbm.at[0].at[pl.ds(0, su)],
                                  sos[b]).wait()

        def add_pos(b):
            su = SU[b % 2]
            t0 = (b % 2) * SU[0]

            @plsc.parallel_loop(0, su, 1, unroll=4)
            def row(i):
                for d in range(D // 16):
                    s0 = pl.ds(d * 16, 16)
                    rows_v[b, i, s0] = rows_v[b, i, s0] + pos_v[t0 + i, s0]

        for b in range(PF):
            start_gather(0, b)

        def body(k, carry):
            for b in range(NB):
                u = NB * k + b
                pb = (b + PF) % NB
                have_next = u + PF < nu
                if b + PF < NB:
                    can_wait = jnp.logical_and(have_next, k >= 1)
                else:
                    can_wait = have_next

                @pl.when(can_wait)
                def _():
                    wait_out(pb)

                @pl.when(have_next)
                def _():
                    start_gather(k + (b + PF) // NB, pb)

                wait_gather(b)
                add_pos(b)
                start_out(k, b)
            return carry

        lax.fori_loop(0, nu // NB, body, 0)
        for b in range(NB):
            wait_out(b)

    return emb


def kernel(x, token_table, pos_table):
    B, T = x.shape
    xf = x.reshape(B * T).astype(jnp.int32)
    return _build(B, T)(xf, token_table, pos_table)
